# Initial kernel scaffold; baseline (speedup 1.0000x reference)
import jax
import jax.numpy as jnp
from jax.experimental import pallas as pl


def kernel(xyz, seg_gt, params):
    raise NotImplementedError("write your pallas kernel here")


# speedup vs baseline: 8.9741x; 8.9741x over previous
import functools

import numpy as np
import jax
import jax.numpy as jnp
from jax.experimental import pallas as pl
from jax.experimental.pallas import tpu as pltpu

_INV_BN = float(1.0 / np.sqrt(1.0 + 1e-5))


def _fps_body(x_ref, y_ref, z_ref, out_ref, *, npoint, n):
    x = x_ref[...]
    y = y_ref[...]
    z = z_ref[...]
    sub = jax.lax.broadcasted_iota(jnp.int32, x.shape, 0)
    lane = jax.lax.broadcasted_iota(jnp.int32, x.shape, 1)
    flat = sub * x.shape[1] + lane

    def body(i, carry):
        dist, f = carry
        out_ref[pl.ds(i, 1), :] = f.reshape(1, 1)
        sel = flat == f
        cx = jnp.sum(jnp.where(sel, x, 0.0))
        cy = jnp.sum(jnp.where(sel, y, 0.0))
        cz = jnp.sum(jnp.where(sel, z, 0.0))
        dx = x - cx
        dy = y - cy
        dz = z - cz
        d = dx * dx + dy * dy + dz * dz
        dist = jnp.minimum(dist, d)
        m = jnp.max(dist)
        f2 = jnp.min(jnp.where(dist == m, flat, n)).astype(jnp.int32)
        return dist, f2

    dist0 = jnp.full(x.shape, 1e10, jnp.float32)
    jax.lax.fori_loop(0, npoint, body, (dist0, jnp.int32(0)))


def _fps(xyz_t, npoint):
    n = xyz_t.shape[0]
    cols = 128
    rows = n // cols
    x = xyz_t[:, 0].reshape(rows, cols)
    y = xyz_t[:, 1].reshape(rows, cols)
    z = xyz_t[:, 2].reshape(rows, cols)
    out = pl.pallas_call(
        functools.partial(_fps_body, npoint=npoint, n=n),
        out_shape=jax.ShapeDtypeStruct((npoint, 1), jnp.int32),
    )(x, y, z)
    return out[:, 0]


def _ballq_body(nx_ref, xyzt_ref, out_ref, *, n, r2, nsample):
    nx = nx_ref[...]
    xyzt = xyzt_ref[...]
    mm = jnp.dot(nx, xyzt, preferred_element_type=jnp.float32)
    sq = -2.0 * mm + jnp.sum(nx * nx, axis=1, keepdims=True)
    sq = sq + jnp.sum(xyzt * xyzt, axis=0, keepdims=True)
    mask = sq <= r2
    c = mask.astype(jnp.float32)
    lane_n = jax.lax.broadcasted_iota(jnp.int32, c.shape, 1)
    sh = 1
    while sh < n:
        rolled = pltpu.roll(c, sh, axis=1)
        c = c + jnp.where(lane_n >= sh, rolled, 0.0)
        sh *= 2
    sb = nx.shape[0]
    lane64 = jax.lax.broadcasted_iota(jnp.int32, (sb, nsample), 1)

    def jbody(j, sel):
        jf = j.astype(jnp.float32)
        col = jnp.sum((c <= jf).astype(jnp.float32), axis=1, keepdims=True)
        return sel + jnp.where(lane64 == j, col, 0.0)

    sel = jax.lax.fori_loop(0, nsample, jbody,
                            jnp.zeros((sb, nsample), jnp.float32))
    first = sel[:, 0:1]
    sel = jnp.where(sel >= float(n), first, sel)
    out_ref[...] = sel.astype(jnp.int32)


def _ballq(new_xyz, xyz_t, radius, nsample, s_blk):
    s = new_xyz.shape[0]
    n = xyz_t.shape[0]
    xyzt = xyz_t.T
    return pl.pallas_call(
        functools.partial(_ballq_body, n=n, r2=float(radius * radius),
                          nsample=nsample),
        grid=(s // s_blk,),
        in_specs=[pl.BlockSpec((s_blk, 3), lambda i: (i, 0)),
                  pl.BlockSpec((3, n), lambda i: (0, 0))],
        out_specs=pl.BlockSpec((s_blk, nsample), lambda i: (i, 0)),
        out_shape=jax.ShapeDtypeStruct((s, nsample), jnp.int32),
    )(new_xyz, xyzt)


def _sa_mlp_body(*refs, nlayers, nsample, s_blk):
    x_ref = refs[0]
    out_ref = refs[-1]
    wb = refs[1:-1]
    h = x_ref[0]
    for l in range(nlayers):
        w = wb[2 * l][...]
        b = wb[2 * l + 1][...]
        h = jnp.dot(w, h, preferred_element_type=jnp.float32) + b
        h = jnp.maximum(h, 0.0)
    m = h[:, 0:s_blk]
    for j in range(1, nsample):
        m = jnp.maximum(m, h[:, j * s_blk:(j + 1) * s_blk])
    out_ref[0] = m


def _sa_mlp(x, layers, nsample, s_blk):
    nblk, ci, cols = x.shape
    args = [x]
    in_specs = [pl.BlockSpec((1, ci, cols), lambda i: (i, 0, 0))]
    for (w, b) in layers:
        args += [w, b.reshape(-1, 1)]
        in_specs += [
            pl.BlockSpec(w.shape, lambda i: (0, 0)),
            pl.BlockSpec((w.shape[0], 1), lambda i: (0, 0)),
        ]
    co = layers[-1][0].shape[0]
    out = pl.pallas_call(
        functools.partial(_sa_mlp_body, nlayers=len(layers),
                          nsample=nsample, s_blk=s_blk),
        grid=(nblk,),
        in_specs=in_specs,
        out_specs=pl.BlockSpec((1, co, s_blk), lambda i: (i, 0, 0)),
        out_shape=jax.ShapeDtypeStruct((nblk, co, s_blk), jnp.float32),
    )(*args)
    return out.transpose(1, 0, 2).reshape(co, nblk * s_blk)


def _to_blocks(npts, s_blk):
    s, ns, c = npts.shape
    g = npts.reshape(s // s_blk, s_blk, ns, c)
    return g.transpose(0, 3, 2, 1).reshape(s // s_blk, c, ns * s_blk)


def _mlp_body(*refs, nlayers):
    x_ref = refs[0]
    out_ref = refs[-1]
    wb = refs[1:-1]
    h = x_ref[...]
    for l in range(nlayers):
        h = jnp.dot(wb[2 * l][...], h, preferred_element_type=jnp.float32)
        h = h + wb[2 * l + 1][...]
        h = jnp.maximum(h, 0.0)
    out_ref[...] = h


def _mlp(x, layers, c_blk):
    ci, n = x.shape
    args = [x]
    in_specs = [pl.BlockSpec((ci, c_blk), lambda i: (0, i))]
    for (w, b) in layers:
        args += [w, b.reshape(-1, 1)]
        in_specs += [
            pl.BlockSpec(w.shape, lambda i: (0, 0)),
            pl.BlockSpec((w.shape[0], 1), lambda i: (0, 0)),
        ]
    co = layers[-1][0].shape[0]
    return pl.pallas_call(
        functools.partial(_mlp_body, nlayers=len(layers)),
        grid=(n // c_blk,),
        in_specs=in_specs,
        out_specs=pl.BlockSpec((co, c_blk), lambda i: (0, i)),
        out_shape=jax.ShapeDtypeStruct((co, n), jnp.float32),
    )(*args)


def _sa3_body(*refs, nlayers):
    x_ref = refs[0]
    out_ref = refs[-1]
    wb = refs[1:-1]
    h = x_ref[...]
    for l in range(nlayers):
        h = jnp.dot(wb[2 * l][...], h, preferred_element_type=jnp.float32)
        h = h + wb[2 * l + 1][...]
        h = jnp.maximum(h, 0.0)
    out_ref[...] = jnp.max(h, axis=1, keepdims=True)


def _sa3(x, layers):
    args = [x]
    for (w, b) in layers:
        args += [w, b.reshape(-1, 1)]
    co = layers[-1][0].shape[0]
    return pl.pallas_call(
        functools.partial(_sa3_body, nlayers=len(layers)),
        out_shape=jax.ShapeDtypeStruct((co, 1), jnp.float32),
    )(*args)


def _interp_body(x1_ref, x2t_ref, p2_ref, out_ref, *, s2):
    x1 = x1_ref[...]
    x2t = x2t_ref[...]
    p2 = p2_ref[...]
    mm = jnp.dot(x1, x2t, preferred_element_type=jnp.float32)
    sq = -2.0 * mm + jnp.sum(x1 * x1, axis=1, keepdims=True)
    sq = sq + jnp.sum(x2t * x2t, axis=0, keepdims=True)
    nb = x1.shape[0]
    lane = jax.lax.broadcasted_iota(jnp.int32, (nb, s2), 1)
    cur = sq
    ds = []
    isel = []
    for _ in range(3):
        m = jnp.min(cur, axis=1, keepdims=True)
        ik = jnp.min(jnp.where(cur == m, lane, s2), axis=1, keepdims=True)
        ds.append(m)
        isel.append(ik)
        cur = jnp.where(lane == ik, 1e30, cur)
    r = [1.0 / (d + 1e-8) for d in ds]
    norm = r[0] + r[1] + r[2]
    w = jnp.zeros((nb, s2), jnp.float32)
    for k in range(3):
        w = w + jnp.where(lane == isel[k], r[k] / norm, 0.0)
    out_ref[...] = jnp.dot(w, p2, preferred_element_type=jnp.float32)


def _interp(xyz1, xyz2, p2, n_blk):
    n1 = xyz1.shape[0]
    s2, c2 = p2.shape
    x2t = xyz2.T
    return pl.pallas_call(
        functools.partial(_interp_body, s2=s2),
        grid=(n1 // n_blk,),
        in_specs=[pl.BlockSpec((n_blk, 3), lambda i: (i, 0)),
                  pl.BlockSpec((3, s2), lambda i: (0, 0)),
                  pl.BlockSpec((s2, c2), lambda i: (0, 0))],
        out_specs=pl.BlockSpec((n_blk, c2), lambda i: (i, 0)),
        out_shape=jax.ShapeDtypeStruct((n1, c2), jnp.float32),
    )(xyz1, x2t, p2)


def _head_body(xyz_ref, seg_ref, p_ref, w0a_ref, w0b_ref, w0c_ref, b0_ref,
               w1a_ref, w1b_ref, w1c_ref, b1_ref,
               w2a_ref, w2b_ref, w2c_ref, b2_ref, out_ref):
    a = xyz_ref[...]
    s = seg_ref[...]
    p = p_ref[...]
    h1 = (jnp.dot(w0a_ref[...], a, preferred_element_type=jnp.float32)
          + jnp.dot(w0b_ref[...], s, preferred_element_type=jnp.float32)
          + jnp.dot(w0c_ref[...], p, preferred_element_type=jnp.float32)
          + b0_ref[...])
    h2 = (jnp.dot(w1a_ref[...], a, preferred_element_type=jnp.float32)
          + jnp.dot(w1b_ref[...], s, preferred_element_type=jnp.float32)
          + jnp.dot(w1c_ref[...], h1, preferred_element_type=jnp.float32)
          + b1_ref[...])
    h3 = (jnp.dot(w2a_ref[...], a, preferred_element_type=jnp.float32)
          + jnp.dot(w2b_ref[...], s, preferred_element_type=jnp.float32)
          + jnp.dot(w2c_ref[...], h2, preferred_element_type=jnp.float32)
          + b2_ref[...])
    out_ref[...] = h3


def _head(xyz0, seg, p0, c00, c10, c12, c_blk):
    n = xyz0.shape[1]
    splits = []
    for conv, nfeat in ((c00, 128), (c10, 128), (c12, 64)):
        w = conv['W']
        splits += [w[:, 0:3], w[:, 3:19], w[:, 19:19 + nfeat],
                   conv['b'].reshape(-1, 1)]
    args = [xyz0, seg, p0] + splits
    in_specs = [pl.BlockSpec((3, c_blk), lambda i: (0, i)),
                pl.BlockSpec((16, c_blk), lambda i: (0, i)),
                pl.BlockSpec((128, c_blk), lambda i: (0, i))]
    for a in splits:
        in_specs.append(pl.BlockSpec(a.shape, lambda i: (0, 0)))
    return pl.pallas_call(
        _head_body,
        grid=(n // c_blk,),
        in_specs=in_specs,
        out_specs=pl.BlockSpec((1, c_blk), lambda i: (0, i)),
        out_shape=jax.ShapeDtypeStruct((1, n), jnp.float32),
    )(*args)


def _fold(layer):
    g = layer['gamma'] * _INV_BN
    return layer['W'] * g[:, None], layer['b'] * g + layer['beta']


def kernel(xyz, seg_gt, params):
    xyz0 = xyz[0]
    seg = seg_gt[0]
    xyz_t = xyz0.T
    n = xyz0.shape[1]

    sa1_l = [_fold(l) for l in params['sa1']]
    sa2_l = [_fold(l) for l in params['sa2']]
    sa3_l = [_fold(l) for l in params['sa3']]
    fp3_l = [_fold(l) for l in params['fp3']]
    fp2_l = [_fold(l) for l in params['fp2']]
    fp1_l = [_fold(l) for l in params['fp1']]

    pts0_t = jnp.concatenate([xyz0, seg], axis=0).T
    fps1 = _fps(xyz_t, 4096)
    new_xyz1 = xyz_t[fps1]
    idx1 = _ballq(new_xyz1, xyz_t, 0.2, 64, 16)
    g_xyz1 = xyz_t[idx1] - new_xyz1[:, None, :]
    g_pts1 = pts0_t[idx1]
    npts1 = jnp.concatenate([g_xyz1, g_pts1], axis=-1)
    l1_points = _sa_mlp(_to_blocks(npts1, 64), sa1_l, 64, 64)

    feats1_t = l1_points.T
    fps2 = _fps(new_xyz1, 1024)
    new_xyz2 = new_xyz1[fps2]
    idx2 = _ballq(new_xyz2, new_xyz1, 0.4, 64, 16)
    g_xyz2 = new_xyz1[idx2] - new_xyz2[:, None, :]
    g_pts2 = feats1_t[idx2]
    npts2 = jnp.concatenate([g_xyz2, g_pts2], axis=-1)
    l2_points = _sa_mlp(_to_blocks(npts2, 64), sa2_l, 64, 64)

    x3 = jnp.concatenate([new_xyz2.T, l2_points], axis=0)
    l3_points = _sa3(x3, sa3_l)

    interp3 = jnp.broadcast_to(l3_points, (1024, 1024))
    f3_in = jnp.concatenate([l2_points, interp3], axis=0)
    l2_new = _mlp(f3_in, fp3_l, 1024)

    interp2 = _interp(new_xyz1, new_xyz2, l2_new.T, 64)
    f2_in = jnp.concatenate([l1_points, interp2.T], axis=0)
    l1_new = _mlp(f2_in, fp2_l, 2048)

    p1 = jnp.concatenate([xyz0, xyz0], axis=0)
    interp1 = _interp(xyz_t, new_xyz1, l1_new.T, 64)
    f1_in = jnp.concatenate([p1, interp1.T], axis=0)
    l0_new = _mlp(f1_in, fp1_l, 2048)

    out = _head(xyz0, seg, l0_new, params['conv00'], params['conv10'],
                params['conv12'], 2048)
    return out[None]


# speedup vs baseline: 8.9826x; 1.0009x over previous
import functools

import numpy as np
import jax
import jax.numpy as jnp
from jax.experimental import pallas as pl
from jax.experimental.pallas import tpu as pltpu

_INV_BN = float(1.0 / np.sqrt(1.0 + 1e-5))


def _fps_body(x_ref, y_ref, z_ref, out_ref, *, npoint, n):
    x = x_ref[...]
    y = y_ref[...]
    z = z_ref[...]
    sub = jax.lax.broadcasted_iota(jnp.int32, x.shape, 0)
    lane = jax.lax.broadcasted_iota(jnp.int32, x.shape, 1)
    flat = sub * x.shape[1] + lane

    def body(i, carry):
        dist, f = carry
        out_ref[pl.ds(i, 1), :] = f.reshape(1, 1)
        r = f // x.shape[1]
        cl = f % x.shape[1]
        lane_row = jax.lax.broadcasted_iota(jnp.int32, (1, x.shape[1]), 1)
        pick = lane_row == cl
        cx = jnp.sum(jnp.where(pick, x_ref[pl.ds(r, 1), :], 0.0))
        cy = jnp.sum(jnp.where(pick, y_ref[pl.ds(r, 1), :], 0.0))
        cz = jnp.sum(jnp.where(pick, z_ref[pl.ds(r, 1), :], 0.0))
        dx = x - cx
        dy = y - cy
        dz = z - cz
        d = dx * dx + dy * dy + dz * dz
        dist = jnp.minimum(dist, d)
        m = jnp.max(dist)
        f2 = jnp.min(jnp.where(dist == m, flat, n)).astype(jnp.int32)
        return dist, f2

    dist0 = jnp.full(x.shape, 1e10, jnp.float32)
    jax.lax.fori_loop(0, npoint, body, (dist0, jnp.int32(0)))


def _fps(xyz_t, npoint):
    n = xyz_t.shape[0]
    cols = 128
    rows = n // cols
    x = xyz_t[:, 0].reshape(rows, cols)
    y = xyz_t[:, 1].reshape(rows, cols)
    z = xyz_t[:, 2].reshape(rows, cols)
    out = pl.pallas_call(
        functools.partial(_fps_body, npoint=npoint, n=n),
        out_shape=jax.ShapeDtypeStruct((npoint, 1), jnp.int32),
    )(x, y, z)
    return out[:, 0]


def _ballq_body(nx_ref, xyzt_ref, out_ref, *, n, r2, nsample):
    nx = nx_ref[...]
    xyzt = xyzt_ref[...]
    mm = jnp.dot(nx, xyzt, preferred_element_type=jnp.float32)
    sq = -2.0 * mm + jnp.sum(nx * nx, axis=1, keepdims=True)
    sq = sq + jnp.sum(xyzt * xyzt, axis=0, keepdims=True)
    mask = sq <= r2
    c = mask.astype(jnp.float32)
    lane_n = jax.lax.broadcasted_iota(jnp.int32, c.shape, 1)
    sh = 1
    while sh < n:
        rolled = pltpu.roll(c, sh, axis=1)
        c = c + jnp.where(lane_n >= sh, rolled, 0.0)
        sh *= 2
    sb = nx.shape[0]
    lane64 = jax.lax.broadcasted_iota(jnp.int32, (sb, nsample), 1)

    def jbody(j, sel):
        jf = j.astype(jnp.float32)
        col = jnp.sum((c <= jf).astype(jnp.float32), axis=1, keepdims=True)
        return sel + jnp.where(lane64 == j, col, 0.0)

    sel = jax.lax.fori_loop(0, nsample, jbody,
                            jnp.zeros((sb, nsample), jnp.float32))
    first = sel[:, 0:1]
    sel = jnp.where(sel >= float(n), first, sel)
    out_ref[...] = sel.astype(jnp.int32)


def _ballq(new_xyz, xyz_t, radius, nsample, s_blk):
    s = new_xyz.shape[0]
    n = xyz_t.shape[0]
    xyzt = xyz_t.T
    return pl.pallas_call(
        functools.partial(_ballq_body, n=n, r2=float(radius * radius),
                          nsample=nsample),
        grid=(s // s_blk,),
        in_specs=[pl.BlockSpec((s_blk, 3), lambda i: (i, 0)),
                  pl.BlockSpec((3, n), lambda i: (0, 0))],
        out_specs=pl.BlockSpec((s_blk, nsample), lambda i: (i, 0)),
        out_shape=jax.ShapeDtypeStruct((s, nsample), jnp.int32),
    )(new_xyz, xyzt)


def _sa_mlp_body(*refs, nlayers, nsample, s_blk):
    x_ref = refs[0]
    out_ref = refs[-1]
    wb = refs[1:-1]
    h = x_ref[0]
    for l in range(nlayers):
        w = wb[2 * l][...]
        b = wb[2 * l + 1][...]
        h = jnp.dot(w, h, preferred_element_type=jnp.float32) + b
        h = jnp.maximum(h, 0.0)
    m = h[:, 0:s_blk]
    for j in range(1, nsample):
        m = jnp.maximum(m, h[:, j * s_blk:(j + 1) * s_blk])
    out_ref[0] = m


def _sa_mlp(x, layers, nsample, s_blk):
    nblk, ci, cols = x.shape
    args = [x]
    in_specs = [pl.BlockSpec((1, ci, cols), lambda i: (i, 0, 0))]
    for (w, b) in layers:
        args += [w, b.reshape(-1, 1)]
        in_specs += [
            pl.BlockSpec(w.shape, lambda i: (0, 0)),
            pl.BlockSpec((w.shape[0], 1), lambda i: (0, 0)),
        ]
    co = layers[-1][0].shape[0]
    out = pl.pallas_call(
        functools.partial(_sa_mlp_body, nlayers=len(layers),
                          nsample=nsample, s_blk=s_blk),
        grid=(nblk,),
        in_specs=in_specs,
        out_specs=pl.BlockSpec((1, co, s_blk), lambda i: (i, 0, 0)),
        out_shape=jax.ShapeDtypeStruct((nblk, co, s_blk), jnp.float32),
    )(*args)
    return out.transpose(1, 0, 2).reshape(co, nblk * s_blk)


def _to_blocks(npts, s_blk):
    s, ns, c = npts.shape
    g = npts.reshape(s // s_blk, s_blk, ns, c)
    return g.transpose(0, 3, 2, 1).reshape(s // s_blk, c, ns * s_blk)


def _mlp_body(*refs, nlayers):
    x_ref = refs[0]
    out_ref = refs[-1]
    wb = refs[1:-1]
    h = x_ref[...]
    for l in range(nlayers):
        h = jnp.dot(wb[2 * l][...], h, preferred_element_type=jnp.float32)
        h = h + wb[2 * l + 1][...]
        h = jnp.maximum(h, 0.0)
    out_ref[...] = h


def _mlp(x, layers, c_blk):
    ci, n = x.shape
    args = [x]
    in_specs = [pl.BlockSpec((ci, c_blk), lambda i: (0, i))]
    for (w, b) in layers:
        args += [w, b.reshape(-1, 1)]
        in_specs += [
            pl.BlockSpec(w.shape, lambda i: (0, 0)),
            pl.BlockSpec((w.shape[0], 1), lambda i: (0, 0)),
        ]
    co = layers[-1][0].shape[0]
    return pl.pallas_call(
        functools.partial(_mlp_body, nlayers=len(layers)),
        grid=(n // c_blk,),
        in_specs=in_specs,
        out_specs=pl.BlockSpec((co, c_blk), lambda i: (0, i)),
        out_shape=jax.ShapeDtypeStruct((co, n), jnp.float32),
    )(*args)


def _sa3_body(*refs, nlayers):
    x_ref = refs[0]
    out_ref = refs[-1]
    wb = refs[1:-1]
    h = x_ref[...]
    for l in range(nlayers):
        h = jnp.dot(wb[2 * l][...], h, preferred_element_type=jnp.float32)
        h = h + wb[2 * l + 1][...]
        h = jnp.maximum(h, 0.0)
    out_ref[...] = jnp.max(h, axis=1, keepdims=True)


def _sa3(x, layers):
    args = [x]
    for (w, b) in layers:
        args += [w, b.reshape(-1, 1)]
    co = layers[-1][0].shape[0]
    return pl.pallas_call(
        functools.partial(_sa3_body, nlayers=len(layers)),
        out_shape=jax.ShapeDtypeStruct((co, 1), jnp.float32),
    )(*args)


def _interp_body(x1_ref, x2t_ref, p2_ref, out_ref, *, s2):
    x1 = x1_ref[...]
    x2t = x2t_ref[...]
    p2 = p2_ref[...]
    mm = jnp.dot(x1, x2t, preferred_element_type=jnp.float32)
    sq = -2.0 * mm + jnp.sum(x1 * x1, axis=1, keepdims=True)
    sq = sq + jnp.sum(x2t * x2t, axis=0, keepdims=True)
    nb = x1.shape[0]
    lane = jax.lax.broadcasted_iota(jnp.int32, (nb, s2), 1)
    cur = sq
    ds = []
    isel = []
    for _ in range(3):
        m = jnp.min(cur, axis=1, keepdims=True)
        ik = jnp.min(jnp.where(cur == m, lane, s2), axis=1, keepdims=True)
        ds.append(m)
        isel.append(ik)
        cur = jnp.where(lane == ik, 1e30, cur)
    r = [1.0 / (d + 1e-8) for d in ds]
    norm = r[0] + r[1] + r[2]
    w = jnp.zeros((nb, s2), jnp.float32)
    for k in range(3):
        w = w + jnp.where(lane == isel[k], r[k] / norm, 0.0)
    out_ref[...] = jnp.dot(w, p2, preferred_element_type=jnp.float32)


def _interp(xyz1, xyz2, p2, n_blk):
    n1 = xyz1.shape[0]
    s2, c2 = p2.shape
    x2t = xyz2.T
    return pl.pallas_call(
        functools.partial(_interp_body, s2=s2),
        grid=(n1 // n_blk,),
        in_specs=[pl.BlockSpec((n_blk, 3), lambda i: (i, 0)),
                  pl.BlockSpec((3, s2), lambda i: (0, 0)),
                  pl.BlockSpec((s2, c2), lambda i: (0, 0))],
        out_specs=pl.BlockSpec((n_blk, c2), lambda i: (i, 0)),
        out_shape=jax.ShapeDtypeStruct((n1, c2), jnp.float32),
    )(xyz1, x2t, p2)


def _head_body(xyz_ref, seg_ref, p_ref, w0a_ref, w0b_ref, w0c_ref, b0_ref,
               w1a_ref, w1b_ref, w1c_ref, b1_ref,
               w2a_ref, w2b_ref, w2c_ref, b2_ref, out_ref):
    a = xyz_ref[...]
    s = seg_ref[...]
    p = p_ref[...]
    h1 = (jnp.dot(w0a_ref[...], a, preferred_element_type=jnp.float32)
          + jnp.dot(w0b_ref[...], s, preferred_element_type=jnp.float32)
          + jnp.dot(w0c_ref[...], p, preferred_element_type=jnp.float32)
          + b0_ref[...])
    h2 = (jnp.dot(w1a_ref[...], a, preferred_element_type=jnp.float32)
          + jnp.dot(w1b_ref[...], s, preferred_element_type=jnp.float32)
          + jnp.dot(w1c_ref[...], h1, preferred_element_type=jnp.float32)
          + b1_ref[...])
    h3 = (jnp.dot(w2a_ref[...], a, preferred_element_type=jnp.float32)
          + jnp.dot(w2b_ref[...], s, preferred_element_type=jnp.float32)
          + jnp.dot(w2c_ref[...], h2, preferred_element_type=jnp.float32)
          + b2_ref[...])
    out_ref[...] = h3


def _head(xyz0, seg, p0, c00, c10, c12, c_blk):
    n = xyz0.shape[1]
    splits = []
    for conv, nfeat in ((c00, 128), (c10, 128), (c12, 64)):
        w = conv['W']
        splits += [w[:, 0:3], w[:, 3:19], w[:, 19:19 + nfeat],
                   conv['b'].reshape(-1, 1)]
    args = [xyz0, seg, p0] + splits
    in_specs = [pl.BlockSpec((3, c_blk), lambda i: (0, i)),
                pl.BlockSpec((16, c_blk), lambda i: (0, i)),
                pl.BlockSpec((128, c_blk), lambda i: (0, i))]
    for a in splits:
        in_specs.append(pl.BlockSpec(a.shape, lambda i: (0, 0)))
    return pl.pallas_call(
        _head_body,
        grid=(n // c_blk,),
        in_specs=in_specs,
        out_specs=pl.BlockSpec((1, c_blk), lambda i: (0, i)),
        out_shape=jax.ShapeDtypeStruct((1, n), jnp.float32),
    )(*args)


def _fold(layer):
    g = layer['gamma'] * _INV_BN
    return layer['W'] * g[:, None], layer['b'] * g + layer['beta']


def kernel(xyz, seg_gt, params):
    xyz0 = xyz[0]
    seg = seg_gt[0]
    xyz_t = xyz0.T
    n = xyz0.shape[1]

    sa1_l = [_fold(l) for l in params['sa1']]
    sa2_l = [_fold(l) for l in params['sa2']]
    sa3_l = [_fold(l) for l in params['sa3']]
    fp3_l = [_fold(l) for l in params['fp3']]
    fp2_l = [_fold(l) for l in params['fp2']]
    fp1_l = [_fold(l) for l in params['fp1']]

    pts0_t = jnp.concatenate([xyz0, seg], axis=0).T
    fps1 = _fps(xyz_t, 4096)
    new_xyz1 = xyz_t[fps1]
    idx1 = _ballq(new_xyz1, xyz_t, 0.2, 64, 16)
    g_xyz1 = xyz_t[idx1] - new_xyz1[:, None, :]
    g_pts1 = pts0_t[idx1]
    npts1 = jnp.concatenate([g_xyz1, g_pts1], axis=-1)
    l1_points = _sa_mlp(_to_blocks(npts1, 64), sa1_l, 64, 64)

    feats1_t = l1_points.T
    fps2 = _fps(new_xyz1, 1024)
    new_xyz2 = new_xyz1[fps2]
    idx2 = _ballq(new_xyz2, new_xyz1, 0.4, 64, 16)
    g_xyz2 = new_xyz1[idx2] - new_xyz2[:, None, :]
    g_pts2 = feats1_t[idx2]
    npts2 = jnp.concatenate([g_xyz2, g_pts2], axis=-1)
    l2_points = _sa_mlp(_to_blocks(npts2, 64), sa2_l, 64, 64)

    x3 = jnp.concatenate([new_xyz2.T, l2_points], axis=0)
    l3_points = _sa3(x3, sa3_l)

    interp3 = jnp.broadcast_to(l3_points, (1024, 1024))
    f3_in = jnp.concatenate([l2_points, interp3], axis=0)
    l2_new = _mlp(f3_in, fp3_l, 1024)

    interp2 = _interp(new_xyz1, new_xyz2, l2_new.T, 64)
    f2_in = jnp.concatenate([l1_points, interp2.T], axis=0)
    l1_new = _mlp(f2_in, fp2_l, 2048)

    p1 = jnp.concatenate([xyz0, xyz0], axis=0)
    interp1 = _interp(xyz_t, new_xyz1, l1_new.T, 64)
    f1_in = jnp.concatenate([p1, interp1.T], axis=0)
    l0_new = _mlp(f1_in, fp1_l, 2048)

    out = _head(xyz0, seg, l0_new, params['conv00'], params['conv10'],
                params['conv12'], 2048)
    return out[None]
